# Initial kernel scaffold; baseline (speedup 1.0000x reference)
import jax
import jax.numpy as jnp
from jax.experimental import pallas as pl


def kernel(g, inputs, W_self1, W_neigh1, b1, W_self2, W_neigh2, b2):
    raise NotImplementedError("write your pallas kernel here")


# speedup vs baseline: 4.5501x; 4.5501x over previous
import functools

import jax
import jax.numpy as jnp
from jax import lax
from jax.experimental import pallas as pl
from jax.experimental.pallas import tpu as pltpu
from jax.experimental.pallas import tpu_sc as plsc

N = 10000
E = 320000
D = 128
NC = 2
NS = 16
NW = NC * NS
CHUNK = 128
CPW = 79
EPW = CPW * CHUNK
E_PAD = EPW * NW
N_ACC = 10240
RPS = N_ACC // NS
DUMMY = N + 8


def _sc_agg_body(compute_deg, src_hbm, dst_hbm, h_hbm, z_hbm, *rest):
    if compute_deg:
        agg_out, deg_out, src_v, dst_v, rows_v, deg_v, acc_sh = rest
    else:
        agg_out, src_v, dst_v, rows_v, acc_sh = rest
        deg_out = deg_v = None

    cid = lax.axis_index("c")
    sid = lax.axis_index("s")
    w = cid * NS + sid

    pltpu.sync_copy(z_hbm, rows_v)
    for r in range(RPS // CHUNK):
        pltpu.sync_copy(rows_v, acc_sh.at[pl.ds(sid * RPS + r * CHUNK, CHUNK)])

    pltpu.sync_copy(src_hbm.at[w], src_v)
    pltpu.sync_copy(dst_hbm.at[w], dst_v)

    if compute_deg:
        z16 = jnp.zeros((16,), jnp.float32)

        def zbody(i, carry):
            deg_v[pl.ds(i * 16, 16)] = z16
            return carry

        lax.fori_loop(0, N_ACC // 16, zbody, 0)

    plsc.subcore_barrier()

    ones16 = jnp.ones((16,), jnp.float32)

    def chunk_body(c, carry):
        pltpu.sync_copy(h_hbm.at[src_v.at[c]], rows_v)
        pltpu.sync_copy(rows_v, acc_sh.at[dst_v.at[c]], add=True)
        if compute_deg:
            for j in range(CHUNK // 16):
                idx16 = dst_v[c, pl.ds(j * 16, 16)]
                plsc.addupdate_scatter(deg_v, [idx16], ones16)
        return carry

    lax.fori_loop(0, CPW, chunk_body, 0)

    plsc.subcore_barrier()

    pltpu.sync_copy(acc_sh.at[pl.ds(sid * RPS, RPS)],
                    agg_out.at[cid, pl.ds(sid * RPS, RPS)])
    if compute_deg:
        pltpu.sync_copy(deg_v, deg_out.at[w])


def _make_sc_agg(compute_deg):
    out_type = [jax.ShapeDtypeStruct((NC, N_ACC, D), jnp.float32)]
    scratch = [
        pltpu.VMEM((CPW, CHUNK), jnp.int32),
        pltpu.VMEM((CPW, CHUNK), jnp.int32),
        pltpu.VMEM((CHUNK, D), jnp.float32),
    ]
    if compute_deg:
        out_type.append(jax.ShapeDtypeStruct((NW, N_ACC), jnp.float32))
        scratch.append(pltpu.VMEM((N_ACC,), jnp.float32))
    scratch.append(pltpu.VMEM_SHARED((N_ACC, D), jnp.float32))

    return pl.kernel(
        functools.partial(_sc_agg_body, compute_deg),
        out_type=out_type,
        mesh=plsc.VectorSubcoreMesh(core_axis_name="c", subcore_axis_name="s"),
        scratch_types=scratch,
        compiler_params=pltpu.CompilerParams(needs_layout_passes=False),
    )


_sc_agg_deg = _make_sc_agg(True)
_sc_agg = _make_sc_agg(False)


def _dense_body(h_ref, a_ref, d_ref, ws_ref, wn_ref, b_ref, o_ref):
    a = a_ref[0] + a_ref[1]
    deg = jnp.sum(d_ref[...], axis=0)
    inv = 1.0 / jnp.maximum(deg, 1.0)
    hn = a * inv[:, None]
    acc = jnp.dot(h_ref[...], ws_ref[...], preferred_element_type=jnp.float32)
    acc = acc + jnp.dot(hn, wn_ref[...], preferred_element_type=jnp.float32)
    o_ref[...] = jnp.maximum(acc + b_ref[0:1, :], 0.0)


_R = 512


def _dense(h, agg, deg_parts, W_self, W_neigh, b2d):
    return pl.pallas_call(
        _dense_body,
        grid=(N_ACC // _R,),
        in_specs=[
            pl.BlockSpec((_R, D), lambda i: (i, 0)),
            pl.BlockSpec((NC, _R, D), lambda i: (0, i, 0)),
            pl.BlockSpec((NW, _R), lambda i: (0, i)),
            pl.BlockSpec((D, D), lambda i: (0, 0)),
            pl.BlockSpec((D, D), lambda i: (0, 0)),
            pl.BlockSpec((8, D), lambda i: (0, 0)),
        ],
        out_specs=pl.BlockSpec((_R, D), lambda i: (i, 0)),
        out_shape=jax.ShapeDtypeStruct((N_ACC, D), jnp.float32),
    )(h, agg, deg_parts, W_self, W_neigh, b2d)


@jax.jit
def kernel(g, inputs, W_self1, W_neigh1, b1, W_self2, W_neigh2, b2):
    pad = E_PAD - E
    src_p = jnp.concatenate(
        [g[0], jnp.zeros((pad,), jnp.int32)]).reshape(NW, CPW, CHUNK)
    dst_p = jnp.concatenate(
        [g[1], jnp.full((pad,), DUMMY, jnp.int32)]).reshape(NW, CPW, CHUNK)
    z = jnp.zeros((CHUNK, D), jnp.float32)
    h0 = jnp.zeros((N_ACC, D), jnp.float32).at[:N, :].set(inputs)
    b1_2 = jnp.broadcast_to(b1, (8, D))
    b2_2 = jnp.broadcast_to(b2, (8, D))

    agg1, deg_parts = _sc_agg_deg(src_p, dst_p, h0, z)
    h1 = _dense(h0, agg1, deg_parts, W_self1, W_neigh1, b1_2)
    (agg2,) = _sc_agg(src_p, dst_p, h1, z)
    h2 = _dense(h1, agg2, deg_parts, W_self2, W_neigh2, b2_2)
    return h2[:N]


# speedup vs baseline: 11.1239x; 2.4447x over previous
import functools

import jax
import jax.numpy as jnp
from jax import lax
from jax.experimental import pallas as pl
from jax.experimental.pallas import tpu as pltpu
from jax.experimental.pallas import tpu_sc as plsc

N = 10000
E = 320000
D = 128
NC = 2
NS = 16
NW = NC * NS
CHUNK = 128
CPW = 80
EPW = CPW * CHUNK
E_PAD = EPW * NW
N_ACC = 10240
RPS = N_ACC // NS
DUMMY = N + 8


def _sc_agg_body(compute_deg, g_hbm, h_hbm, z_hbm, *rest):
    if compute_deg:
        (agg_out, deg_out, idx_a, idx_b, buf_a, buf_b, deg_v, acc_sh,
         sem_a, sem_b) = rest
    else:
        agg_out, idx_a, idx_b, buf_a, buf_b, acc_sh, sem_a, sem_b = rest
        deg_out = deg_v = None

    cid = lax.axis_index("c")
    sid = lax.axis_index("s")
    w = cid * NS + sid

    pltpu.sync_copy(z_hbm, buf_a)
    for r in range(RPS // CHUNK):
        pltpu.sync_copy(buf_a, acc_sh.at[pl.ds(sid * RPS + r * CHUNK, CHUNK)])

    if compute_deg:
        z16 = jnp.zeros((16,), jnp.float32)

        def zbody(i, carry):
            deg_v[pl.ds(i * 16, 16)] = z16
            return carry

        lax.fori_loop(0, N_ACC // 16, zbody, 0)

    plsc.subcore_barrier()

    ones16 = jnp.ones((16,), jnp.float32)

    def deg_update(idx_v):
        if compute_deg:
            for j in range(CHUNK // 16):
                idx16 = idx_v[1, pl.ds(j * 16, 16)]
                plsc.addupdate_scatter(deg_v, [idx16], ones16)

    pltpu.sync_copy(g_hbm.at[w, 0], idx_a)
    pltpu.async_copy(h_hbm.at[idx_a.at[0]], buf_a, sem_a)
    pltpu.sync_copy(g_hbm.at[w, 1], idx_b)

    def pair_body(g, carry):
        c0 = 2 * g
        pltpu.make_async_copy(h_hbm.at[idx_a.at[0]], buf_a, sem_a).wait()
        pltpu.async_copy(h_hbm.at[idx_b.at[0]], buf_b, sem_b)
        pltpu.sync_copy(buf_a, acc_sh.at[idx_a.at[1]], add=True)
        deg_update(idx_a)
        c2 = jnp.minimum(c0 + 2, CPW - 1)
        pltpu.sync_copy(g_hbm.at[w, c2], idx_a)
        pltpu.make_async_copy(h_hbm.at[idx_b.at[0]], buf_b, sem_b).wait()
        pltpu.async_copy(h_hbm.at[idx_a.at[0]], buf_a, sem_a)
        pltpu.sync_copy(buf_b, acc_sh.at[idx_b.at[1]], add=True)
        deg_update(idx_b)
        c3 = jnp.minimum(c0 + 3, CPW - 1)
        pltpu.sync_copy(g_hbm.at[w, c3], idx_b)
        return carry

    lax.fori_loop(0, CPW // 2, pair_body, 0)
    pltpu.make_async_copy(h_hbm.at[idx_a.at[0]], buf_a, sem_a).wait()

    plsc.subcore_barrier()

    pltpu.sync_copy(acc_sh.at[pl.ds(sid * RPS, RPS)],
                    agg_out.at[cid, pl.ds(sid * RPS, RPS)])
    if compute_deg:
        pltpu.sync_copy(deg_v, deg_out.at[w])


def _make_sc_agg(compute_deg):
    out_type = [jax.ShapeDtypeStruct((NC, N_ACC, D), jnp.float32)]
    scratch = [
        pltpu.VMEM((2, CHUNK), jnp.int32),
        pltpu.VMEM((2, CHUNK), jnp.int32),
        pltpu.VMEM((CHUNK, D), jnp.float32),
        pltpu.VMEM((CHUNK, D), jnp.float32),
    ]
    if compute_deg:
        out_type.append(jax.ShapeDtypeStruct((NW, N_ACC), jnp.float32))
        scratch.append(pltpu.VMEM((N_ACC,), jnp.float32))
    scratch.append(pltpu.VMEM_SHARED((N_ACC, D), jnp.float32))
    scratch.append(pltpu.SemaphoreType.DMA)
    scratch.append(pltpu.SemaphoreType.DMA)

    return pl.kernel(
        functools.partial(_sc_agg_body, compute_deg),
        out_type=out_type,
        mesh=plsc.VectorSubcoreMesh(core_axis_name="c", subcore_axis_name="s"),
        scratch_types=scratch,
        compiler_params=pltpu.CompilerParams(needs_layout_passes=False),
    )


_sc_agg_deg = _make_sc_agg(True)
_sc_agg = _make_sc_agg(False)


def _dense_body(h_ref, a_ref, d_ref, ws_ref, wn_ref, b_ref, o_ref):
    a = a_ref[0] + a_ref[1]
    deg = jnp.sum(d_ref[...], axis=0)
    inv = 1.0 / jnp.maximum(deg, 1.0)
    hn = a * inv[:, None]
    acc = jnp.dot(h_ref[...], ws_ref[...], preferred_element_type=jnp.float32)
    acc = acc + jnp.dot(hn, wn_ref[...], preferred_element_type=jnp.float32)
    o_ref[...] = jnp.maximum(acc + b_ref[0:1, :], 0.0)


_R = 512


def _dense(h, agg, deg_parts, W_self, W_neigh, b2d):
    return pl.pallas_call(
        _dense_body,
        grid=(N_ACC // _R,),
        in_specs=[
            pl.BlockSpec((_R, D), lambda i: (i, 0)),
            pl.BlockSpec((NC, _R, D), lambda i: (0, i, 0)),
            pl.BlockSpec((NW, _R), lambda i: (0, i)),
            pl.BlockSpec((D, D), lambda i: (0, 0)),
            pl.BlockSpec((D, D), lambda i: (0, 0)),
            pl.BlockSpec((8, D), lambda i: (0, 0)),
        ],
        out_specs=pl.BlockSpec((_R, D), lambda i: (i, 0)),
        out_shape=jax.ShapeDtypeStruct((N_ACC, D), jnp.float32),
    )(h, agg, deg_parts, W_self, W_neigh, b2d)


@jax.jit
def kernel(g, inputs, W_self1, W_neigh1, b1, W_self2, W_neigh2, b2):
    pad = E_PAD - E
    pad_src = (jnp.arange(pad, dtype=jnp.int32) * 131) % N
    pad_dst = DUMMY + (jnp.arange(pad, dtype=jnp.int32) % 32)
    src_p = jnp.concatenate([g[0], pad_src]).reshape(NW, CPW, CHUNK)
    dst_p = jnp.concatenate([g[1], pad_dst]).reshape(NW, CPW, CHUNK)
    g_p = jnp.stack([src_p, dst_p], axis=2)
    z = jnp.zeros((CHUNK, D), jnp.float32)
    h0 = jnp.zeros((N_ACC, D), jnp.float32).at[:N, :].set(inputs)
    b1_2 = jnp.broadcast_to(b1, (8, D))
    b2_2 = jnp.broadcast_to(b2, (8, D))

    agg1, deg_parts = _sc_agg_deg(g_p, h0, z)
    h1 = _dense(h0, agg1, deg_parts, W_self1, W_neigh1, b1_2)
    (agg2,) = _sc_agg(g_p, h1, z)
    h2 = _dense(h1, agg2, deg_parts, W_self2, W_neigh2, b2_2)
    return h2[:N]
